# baseline (speedup 1.0000x reference)
import functools

import jax
import jax.numpy as jnp
from jax import lax
from jax.experimental import pallas as pl
from jax.experimental.pallas import tpu as pltpu
from jax.experimental.pallas import tpu_sc as plsc

B = 4
S = 8192
D = 768
LANES = 16
NC = 2
NS = 16
NW = NC * NS
ROWS_PER_W = S // NW
K = 64
NCHUNK = ROWS_PER_W // K
DVEC = D // LANES


def _sc_kernel(enc_hbm, tidx_hbm, pos_hbm, out_hbm, idx_v, t0_v, rows_v,
               buf_v, sem):
    wid = lax.axis_index("s") * NC + lax.axis_index("c")
    base = wid * ROWS_PER_W

    pltpu.sync_copy(tidx_hbm.at[pl.ds(0, LANES)], t0_v)
    t0 = lax.gather(
        t0_v[...],
        jnp.zeros((LANES, 1), jnp.int32),
        dimension_numbers=lax.GatherDimensionNumbers(
            offset_dims=(), collapsed_slice_dims=(0,), start_index_map=(0,)),
        slice_sizes=(1,),
        mode=lax.GatherScatterMode.PROMISE_IN_BOUNDS)

    def chunk_body(ci, carry):
        cbase = base + ci * K
        pltpu.sync_copy(tidx_hbm.at[pl.ds(cbase, K)], idx_v)
        for j in range(K // LANES):
            sl = pl.ds(j * LANES, LANES)
            idx_v[sl] = idx_v[sl] - t0
        pltpu.async_copy(pos_hbm.at[idx_v], rows_v, sem).wait()
        for b in range(B):
            pltpu.sync_copy(enc_hbm.at[b, pl.ds(cbase, K)], buf_v)

            def row_body(r, c2):
                for c in range(DVEC):
                    sl = pl.ds(c * LANES, LANES)
                    plsc.addupdate(buf_v.at[r, sl], rows_v[r, sl])
                return c2

            lax.fori_loop(0, K, row_body, 0)
            pltpu.sync_copy(buf_v, out_hbm.at[b, pl.ds(cbase, K)])
        return carry

    lax.fori_loop(0, NCHUNK, chunk_body, 0)


@jax.jit
def _run(enc_inputs, tindex, pos_table):
    mesh = plsc.VectorSubcoreMesh(core_axis_name="c", subcore_axis_name="s")
    kfn = functools.partial(
        pl.kernel,
        mesh=mesh,
        out_type=jax.ShapeDtypeStruct((B, S, D), jnp.float32),
        scratch_types=[
            pltpu.VMEM((K,), jnp.int32),
            pltpu.VMEM((LANES,), jnp.int32),
            pltpu.VMEM((K, D), jnp.float32),
            pltpu.VMEM((K, D), jnp.float32),
            pltpu.SemaphoreType.DMA,
        ],
    )(_sc_kernel)
    return kfn(enc_inputs, tindex, pos_table)


def kernel(enc_inputs, tindex, pos_table):
    return _run(enc_inputs, tindex, pos_table)
